# Initial kernel scaffold; baseline (speedup 1.0000x reference)
import jax
import jax.numpy as jnp
from jax.experimental import pallas as pl


def kernel(feature, adj, emb, W, a_src, a_dst, W_pen, b_pen, W_out, b_out):
    raise NotImplementedError("write your pallas kernel here")


# speedup vs baseline: 18.0021x; 18.0021x over previous
import functools

import jax
import jax.numpy as jnp
from jax import lax
from jax.experimental import pallas as pl
from jax.experimental.pallas import tpu as pltpu
from jax.experimental.pallas import tpu_sc as plsc

N_NODES = 50000
N_PAD = 50176
N_EDGES = 800000
D_EMB = 32
HID = 64
PEN = 64
ALPHA = 0.2

NC, NS = 2, 16
NW = NC * NS
BPW = N_PAD // NW
HALF = N_NODES // 2
TBL = 25088
TPT = TBL // NS
DUMMY = 25080
EPT = N_EDGES // NS
EB = 80
NB = EPT // EB

_mesh = plsc.VectorSubcoreMesh(core_axis_name="c", subcore_axis_name="s")


@functools.partial(
    pl.kernel, mesh=_mesh,
    compiler_params=pltpu.CompilerParams(use_tc_tiling_on_sc=False),
    out_type=jax.ShapeDtypeStruct((N_PAD, D_EMB), jnp.float32),
    scratch_types=[
        pltpu.VMEM((BPW,), jnp.int32),
        pltpu.VMEM((BPW, D_EMB), jnp.float32),
        pltpu.SemaphoreType.DMA,
    ],
)
def _sc_gather(table_hbm, idx_hbm, out_hbm, idx_v, rows_v, sem):
    wid = lax.axis_index("s") * NC + lax.axis_index("c")
    base = wid * BPW
    pltpu.sync_copy(idx_hbm.at[pl.ds(base, BPW)], idx_v)
    pltpu.async_copy(table_hbm.at[idx_v], rows_v, sem).wait()
    pltpu.sync_copy(rows_v, out_hbm.at[pl.ds(base, BPW)])


@functools.partial(
    pl.kernel, mesh=_mesh,
    compiler_params=pltpu.CompilerParams(use_tc_tiling_on_sc=False),
    out_type=jax.ShapeDtypeStruct((NC, 2, TBL, 48), jnp.float32),
    scratch_types=[
        pltpu.VMEM((EB,), jnp.int32),
        pltpu.VMEM((EB,), jnp.int32),
        pltpu.VMEM((EB,), jnp.int32),
        pltpu.VMEM((EB, 64), jnp.float32),
        pltpu.VMEM((EB, 16), jnp.float32),
        pltpu.VMEM((EB, 48), jnp.float32),
        pltpu.VMEM_SHARED((TBL, 48), jnp.float32),
        pltpu.SemaphoreType.DMA,
    ],
)
def _sc_edges(t64a_hbm, t64b_hbm, v16_hbm, adj_hbm, zer_hbm, out_hbm,
              sidx, didx, lidx, gsrc, gdst, orow, shared, sem):
    cid = lax.axis_index("c")
    s = lax.axis_index("s")
    lo = cid * HALF

    lanes = lax.iota(jnp.int32, 16)
    mask4 = jnp.where(lanes < 4, jnp.float32(1.0), jnp.float32(0.0))

    for p in range(2):
        t64_hbm = t64a_hbm if p == 0 else t64b_hbm
        pltpu.sync_copy(zer_hbm, shared.at[pl.ds(s * TPT, TPT)])
        plsc.subcore_barrier()

        def block(b, carry):
            off = s * EPT + b * EB
            pltpu.sync_copy(adj_hbm.at[0, pl.ds(off, EB)], sidx)
            pltpu.sync_copy(adj_hbm.at[1, pl.ds(off, EB)], didx)
            pltpu.async_copy(t64_hbm.at[sidx], gsrc, sem).wait()
            pltpu.async_copy(v16_hbm.at[didx], gdst, sem).wait()

            for j in range(EB // 16):
                d16 = didx[pl.ds(j * 16, 16)]
                inr = (d16 >= lo) & (d16 < lo + HALF)
                lidx[pl.ds(j * 16, 16)] = jnp.where(inr, d16 - lo, DUMMY)

            def _g(vec, idxq):
                return lax.gather(
                    vec, idxq[:, None],
                    lax.GatherDimensionNumbers(
                        offset_dims=(), collapsed_slice_dims=(0,),
                        start_index_map=(0,)),
                    (1,), mode=lax.GatherScatterMode.PROMISE_IN_BOUNDS)

            def edge(i, c2):
                us = gsrc[i, pl.ds(0, 16)]
                vd = gdst[i, pl.ds(0, 16)]
                t = us + vd
                ex = jnp.exp(jnp.maximum(t, ALPHA * t))
                ln = lax.iota(jnp.int32, 16)
                idx4 = jnp.where(ln < 4, ln + 4 * p, 0)
                ex4 = _g(ex, idx4)
                orow[i, pl.ds(0, 16)] = ex4 * mask4
                for r in range(2):
                    h0, h1 = 4 * p + 2 * r, 4 * p + 2 * r + 1
                    idxr = jnp.where(ln < 8, h0, h1)
                    exr = _g(ex, idxr)
                    wh = gsrc[i, pl.ds(16 + r * 16, 16)]
                    orow[i, pl.ds(16 + r * 16, 16)] = exr * wh
                return c2

            lax.fori_loop(0, EB, edge, 0)
            pltpu.sync_copy(orow, shared.at[lidx], add=True)
            return carry

        lax.fori_loop(0, NB, block, 0)

        plsc.subcore_barrier()
        pltpu.sync_copy(shared.at[pl.ds(s * TPT, TPT)],
                        out_hbm.at[cid, p, pl.ds(s * TPT, TPT)])


_PR = 512


def _prep_body(h_ref, wf_ref, as_ref, ad_ref, ta_ref, tb_ref, v_ref):
    wh = jnp.dot(h_ref[...], wf_ref[...], preferred_element_type=jnp.float32)
    u = jnp.dot(wh, as_ref[...], preferred_element_type=jnp.float32)
    v = jnp.dot(wh, ad_ref[...], preferred_element_type=jnp.float32)
    z8 = jnp.zeros((_PR, 8), jnp.float32)
    z16 = jnp.zeros((_PR, 16), jnp.float32)
    ta_ref[...] = jnp.concatenate([u, z8, wh[:, 0:32], z16], axis=1)
    tb_ref[...] = jnp.concatenate([u, z8, wh[:, 32:64], z16], axis=1)
    v_ref[...] = jnp.concatenate([v, z8], axis=1)


def _tc_prep(h, w_flat, a_src_m, a_dst_m):
    return pl.pallas_call(
        _prep_body,
        grid=(N_PAD // _PR,),
        in_specs=[
            pl.BlockSpec((_PR, D_EMB), lambda i: (i, 0)),
            pl.BlockSpec((D_EMB, HID), lambda i: (0, 0)),
            pl.BlockSpec((HID, 8), lambda i: (0, 0)),
            pl.BlockSpec((HID, 8), lambda i: (0, 0)),
        ],
        out_specs=[
            pl.BlockSpec((_PR, 64), lambda i: (i, 0)),
            pl.BlockSpec((_PR, 64), lambda i: (i, 0)),
            pl.BlockSpec((_PR, 16), lambda i: (i, 0)),
        ],
        out_shape=[
            jax.ShapeDtypeStruct((N_PAD, 64), jnp.float32),
            jax.ShapeDtypeStruct((N_PAD, 64), jnp.float32),
            jax.ShapeDtypeStruct((N_PAD, 16), jnp.float32),
        ],
    )(h, w_flat, a_src_m, a_dst_m)


_HR = 400
_HG = N_NODES // _HR


def _head_body(den_ref, msg_ref, r8_ref, wp_ref, bp_ref, wo_ref, bo_ref,
               lat_ref, log_ref):
    pid = pl.program_id(0)
    den = den_ref[...] + 1e-16
    denr = jnp.dot(den, r8_ref[...], preferred_element_type=jnp.float32)
    agg = msg_ref[...] / denr
    h1 = jnp.where(agg > 0, agg, jnp.exp(jnp.minimum(agg, 0.0)) - 1.0)
    pen = jnp.dot(h1, wp_ref[...], preferred_element_type=jnp.float32) + bp_ref[...]
    pen = jnp.where(pen > 0, pen, jnp.exp(jnp.minimum(pen, 0.0)) - 1.0)

    @pl.when(pid == 0)
    def _():
        lat_ref[...] = jnp.zeros((1, PEN), jnp.float32)
        log_ref[...] = jnp.zeros((1, 2), jnp.float32)

    lat_ref[...] += jnp.sum(pen, axis=0, keepdims=True)

    @pl.when(pid == _HG - 1)
    def _():
        latent = lat_ref[...] / jnp.float32(N_NODES)
        lat_ref[...] = latent
        log_ref[...] = jnp.dot(latent, wo_ref[...],
                               preferred_element_type=jnp.float32) + bo_ref[...]


def _tc_head(den, msg, r8, w_pen, b_pen, w_out, b_out):
    return pl.pallas_call(
        _head_body,
        grid=(_HG,),
        in_specs=[
            pl.BlockSpec((_HR, 8), lambda i: (i, 0)),
            pl.BlockSpec((_HR, HID), lambda i: (i, 0)),
            pl.BlockSpec((8, HID), lambda i: (0, 0)),
            pl.BlockSpec((HID, PEN), lambda i: (0, 0)),
            pl.BlockSpec((1, PEN), lambda i: (0, 0)),
            pl.BlockSpec((PEN, 2), lambda i: (0, 0)),
            pl.BlockSpec((1, 2), lambda i: (0, 0)),
        ],
        out_specs=[
            pl.BlockSpec((1, PEN), lambda i: (0, 0)),
            pl.BlockSpec((1, 2), lambda i: (0, 0)),
        ],
        out_shape=[
            jax.ShapeDtypeStruct((1, PEN), jnp.float32),
            jax.ShapeDtypeStruct((1, 2), jnp.float32),
        ],
    )(den, msg, r8, w_pen, b_pen, w_out, b_out)


def _forward(feature, adj, emb, W, a_src, a_dst, W_pen, b_pen, W_out, b_out):
    idxp = jnp.concatenate(
        [feature.astype(jnp.int32),
         jnp.zeros((N_PAD - N_NODES,), jnp.int32)])
    w_flat = W.transpose(1, 0, 2).reshape(D_EMB, HID)
    cols = jnp.arange(HID)
    a_src_m = jnp.zeros((HID, 8), jnp.float32).at[cols, cols // 8].set(
        a_src.reshape(HID))
    a_dst_m = jnp.zeros((HID, 8), jnp.float32).at[cols, cols // 8].set(
        a_dst.reshape(HID))
    r8 = jnp.zeros((8, HID), jnp.float32).at[cols // 8, cols].set(1.0)
    zer = jnp.zeros((TPT, 48), jnp.float32)

    h = _sc_gather(emb, idxp)
    t64a, t64b, v16 = _tc_prep(h, w_flat, a_src_m, a_dst_m)
    acc = _sc_edges(t64a, t64b, v16, adj.astype(jnp.int32), zer)
    den = jnp.concatenate(
        [jnp.concatenate([acc[c, 0, :HALF, 0:4], acc[c, 1, :HALF, 0:4]],
                         axis=1) for c in range(NC)], axis=0)
    msg = jnp.concatenate(
        [jnp.concatenate([acc[c, 0, :HALF, 16:48], acc[c, 1, :HALF, 16:48]],
                         axis=1) for c in range(NC)], axis=0)
    lat, log = _tc_head(den, msg, r8, W_pen, b_pen.reshape(1, PEN),
                        W_out, b_out.reshape(1, 2))
    return lat.reshape(PEN), log.reshape(2)


_jit_forward = jax.jit(_forward)


def kernel(feature, adj, emb, W, a_src, a_dst, W_pen, b_pen, W_out, b_out):
    return _jit_forward(feature, adj, emb, W, a_src, a_dst,
                        W_pen, b_pen, W_out, b_out)
